# baseline (speedup 1.0000x reference)
import functools

import jax
import jax.numpy as jnp
from jax import lax
from jax.experimental import pallas as pl
from jax.experimental.pallas import tpu as pltpu
from jax.experimental.pallas import tpu_sc as plsc

B = 64
H = 224
W = 224
C = 8
WC = W * C
PIX = H * W
N = B * PIX


def _tc_body(x_ref, dx_ref, dy_ref, tab_ref, idx_ref):
    b = pl.program_id(0)
    blk = x_ref[0]
    v = blk[0:222, :] + blk[2:224, :]
    t = (v[:, 0 : WC - 2 * C] + v[:, 2 * C : WC]) * 0.25
    tab_ref[0] = jnp.pad(t, ((1, 1), (C, C)))

    dx = dx_ref[0]
    dy = dy_ref[0]
    xg = lax.broadcasted_iota(jnp.int32, (H, W), 1).astype(jnp.float32)
    yg = lax.broadcasted_iota(jnp.int32, (H, W), 0).astype(jnp.float32)
    xi = jnp.mod(xg + dx, 224.0).astype(jnp.int32)
    yi = jnp.mod(yg + dy, 224.0).astype(jnp.int32)
    xi = jnp.minimum(xi, 223)
    yi = jnp.minimum(yi, 223)
    idx_ref[0] = (b * H + yi) * W + xi


def _tc_stage(x8, dx, dy):
    return pl.pallas_call(
        _tc_body,
        grid=(B,),
        in_specs=[
            pl.BlockSpec((1, H, WC), lambda b: (b, 0, 0)),
            pl.BlockSpec((1, H, W), lambda b: (b, 0, 0)),
            pl.BlockSpec((1, H, W), lambda b: (b, 0, 0)),
        ],
        out_specs=[
            pl.BlockSpec((1, H, WC), lambda b: (b, 0, 0)),
            pl.BlockSpec((1, H, W), lambda b: (b, 0, 0)),
        ],
        out_shape=[
            jax.ShapeDtypeStruct((B, H, WC), jnp.float32),
            jax.ShapeDtypeStruct((B, H, W), jnp.int32),
        ],
    )(x8, dx, dy)


_NC = 2
_NS = 16
_NW = _NC * _NS
SUB = 128
G = 16
NROW = N // SUB
PER_W = NROW // _NW
N_CHUNKS = PER_W // G


def _sc_gather(table, idx2):
    mesh = plsc.VectorSubcoreMesh(core_axis_name="c", subcore_axis_name="s")

    @functools.partial(
        pl.kernel,
        mesh=mesh,
        out_type=jax.ShapeDtypeStruct((NROW, SUB, C), jnp.float32),
        compiler_params=pltpu.CompilerParams(use_tc_tiling_on_sc=False),
        scratch_types=[
            pltpu.VMEM((G, SUB), jnp.int32),
            pltpu.VMEM((G, SUB, C), jnp.float32),
            pltpu.SemaphoreType.DMA,
        ],
    )
    def k(table_hbm, idx_hbm, out_hbm, idx_v, rows_v, sem):
        wid = lax.axis_index("s") * _NC + lax.axis_index("c")
        base = wid * PER_W

        def body(i, _):
            off = base + i * G
            pltpu.sync_copy(idx_hbm.at[pl.ds(off, G)], idx_v)
            descs = [
                pltpu.async_copy(table_hbm.at[idx_v.at[j]], rows_v.at[j], sem)
                for j in range(G)
            ]
            for d in descs:
                d.wait()
            pltpu.sync_copy(rows_v, out_hbm.at[pl.ds(off, G)])
            return 0

        lax.fori_loop(0, N_CHUNKS, body, 0)

    return k(table, idx2)


def kernel(x):
    x8 = jnp.pad(x, ((0, 0), (0, 0), (0, 0), (0, 3))).reshape(B, H, WC)
    dx = x[:, :, :, 3]
    dy = x[:, :, :, 4]
    tab, idx = _tc_stage(x8, dx, dy)
    out8 = _sc_gather(tab.reshape(N, C), idx.reshape(NROW, SUB))
    return out8.reshape(B, H, W, C)[..., 0:3]
